# baseline (speedup 1.0000x reference)
import jax
import jax.numpy as jnp
from jax.experimental import pallas as pl
from jax.experimental.pallas import tpu as pltpu

_N = 32768
_D = 768
_E = 64
_B = 2048
_G = _N // _B


def _router_body(x_ref, wg_ref, tri_ref,
                 g1_ref, idx_ref, loc_ref, me_ref, laux_ref, ent_ref,
                 cnt_ref, meacc_ref, entacc_ref):
    i = pl.program_id(0)

    @pl.when(i == 0)
    def _():
        cnt_ref[...] = jnp.zeros_like(cnt_ref)
        meacc_ref[...] = jnp.zeros_like(meacc_ref)
        entacc_ref[...] = jnp.zeros_like(entacc_ref)

    lt = jax.lax.dot_general(wg_ref[...], x_ref[...],
                             (((1,), (1,)), ((), ())),
                             preferred_element_type=jnp.float32)
    m = jnp.max(lt, axis=0, keepdims=True)
    d = lt - m
    e = jnp.exp(d)
    s = jnp.sum(e, axis=0, keepdims=True)
    t1 = jnp.sum(e * d, axis=0, keepdims=True)
    r = 1.0 / s
    entacc_ref[...] += jnp.log(s) - t1 * r
    g1_ref[...] = r.reshape(1, 1, _B)
    meacc_ref[...] += e * r

    row = jax.lax.broadcasted_iota(jnp.int32, (_E, _B), 0)
    idx = jnp.min(jnp.where(lt == m, row, _E), axis=0, keepdims=True)
    idx_ref[...] = idx.reshape(1, 1, _B)
    onehot = (row == idx).astype(jnp.float32)

    csum = jax.lax.dot_general(onehot.astype(jnp.bfloat16), tri_ref[...],
                               (((1,), (0,)), ((), ())),
                               preferred_element_type=jnp.float32)
    base = cnt_ref[...]
    loc = jnp.sum((csum + base) * onehot, axis=0, keepdims=True) - 1.0
    loc_ref[...] = loc.astype(jnp.int32).reshape(1, 1, _B)
    cnt_ref[...] = base + jax.lax.slice(csum, (0, _B - 1), (_E, _B))

    @pl.when(i == _G - 1)
    def _():
        me = jnp.sum(meacc_ref[...], axis=1, keepdims=True)
        me_ref[...] = me
        laux_ref[...] = (jnp.sum(me * cnt_ref[...], axis=0, keepdims=True)
                         * (_E / (_N * _N)))
        ent_ref[...] = jnp.sum(entacc_ref[...], axis=1, keepdims=True) / _N


def _run(input, wg, interpret=False):
    tri = (jax.lax.broadcasted_iota(jnp.int32, (_B, _B), 0) <=
           jax.lax.broadcasted_iota(jnp.int32, (_B, _B), 1)).astype(jnp.bfloat16)
    g1, idx, loc, me, laux, ent = pl.pallas_call(
        _router_body,
        grid=(_G,),
        in_specs=[
            pl.BlockSpec((_B, _D), lambda i: (i, 0)),
            pl.BlockSpec((_E, _D), lambda i: (0, 0)),
            pl.BlockSpec((_B, _B), lambda i: (0, 0)),
        ],
        out_specs=[
            pl.BlockSpec((1, 1, _B), lambda i: (i, 0, 0)),
            pl.BlockSpec((1, 1, _B), lambda i: (i, 0, 0)),
            pl.BlockSpec((1, 1, _B), lambda i: (i, 0, 0)),
            pl.BlockSpec((_E, 1), lambda i: (0, 0)),
            pl.BlockSpec((1, 1), lambda i: (0, 0)),
            pl.BlockSpec((1, 1), lambda i: (0, 0)),
        ],
        out_shape=[
            jax.ShapeDtypeStruct((_G, 1, _B), jnp.float32),
            jax.ShapeDtypeStruct((_G, 1, _B), jnp.int32),
            jax.ShapeDtypeStruct((_G, 1, _B), jnp.int32),
            jax.ShapeDtypeStruct((_E, 1), jnp.float32),
            jax.ShapeDtypeStruct((1, 1), jnp.float32),
            jax.ShapeDtypeStruct((1, 1), jnp.float32),
        ],
        scratch_shapes=[
            pltpu.VMEM((_E, 1), jnp.float32),
            pltpu.VMEM((_E, _B), jnp.float32),
            pltpu.VMEM((1, _B), jnp.float32),
        ],
        compiler_params=pltpu.CompilerParams(
            dimension_semantics=("arbitrary",),
        ),
        interpret=interpret,
    )(input, wg, tri)
    return (laux.reshape(()), g1.reshape(_N), idx.reshape(_N),
            loc.reshape(_N), ent.reshape(()))


def kernel(input, wg):
    return _run(input, wg)


# speedup vs baseline: 1.0402x; 1.0402x over previous
import jax
import jax.numpy as jnp
from jax.experimental import pallas as pl
from jax.experimental.pallas import tpu as pltpu

_N = 32768
_D = 768
_E = 64
_B = 1024
_G = _N // _B


def _router_body(x_ref, wg_ref, tri_ref,
                 g1_ref, idx_ref, loc_ref, me_ref, laux_ref, ent_ref,
                 cnt_ref, meacc_ref, entacc_ref):
    i = pl.program_id(0)

    @pl.when(i == 0)
    def _():
        cnt_ref[...] = jnp.zeros_like(cnt_ref)
        meacc_ref[...] = jnp.zeros_like(meacc_ref)
        entacc_ref[...] = jnp.zeros_like(entacc_ref)

    lt = jax.lax.dot_general(wg_ref[...], x_ref[...],
                             (((1,), (1,)), ((), ())),
                             preferred_element_type=jnp.float32)
    m = jnp.max(lt, axis=0, keepdims=True)
    d = lt - m
    e = jnp.exp(d)
    s = jnp.sum(e, axis=0, keepdims=True)
    t1 = jnp.sum(e * d, axis=0, keepdims=True)
    r = 1.0 / s
    entacc_ref[...] += jnp.log(s) - t1 * r
    g1_ref[...] = r.reshape(1, 1, _B)
    meacc_ref[...] += e * r

    row = jax.lax.broadcasted_iota(jnp.int32, (_E, _B), 0)
    idx = jnp.min(jnp.where(lt == m, row, _E), axis=0, keepdims=True)
    idx_ref[...] = idx.reshape(1, 1, _B)
    onehot = (row == idx).astype(jnp.float32)

    csum = jax.lax.dot_general(onehot.astype(jnp.bfloat16), tri_ref[...],
                               (((1,), (0,)), ((), ())),
                               preferred_element_type=jnp.float32)
    base = cnt_ref[...]
    loc = jnp.sum((csum + base) * onehot, axis=0, keepdims=True) - 1.0
    loc_ref[...] = loc.astype(jnp.int32).reshape(1, 1, _B)
    cnt_ref[...] = base + jax.lax.slice(csum, (0, _B - 1), (_E, _B))

    @pl.when(i == _G - 1)
    def _():
        me = jnp.sum(meacc_ref[...], axis=1, keepdims=True)
        me_ref[...] = me
        laux_ref[...] = (jnp.sum(me * cnt_ref[...], axis=0, keepdims=True)
                         * (_E / (_N * _N)))
        ent_ref[...] = jnp.sum(entacc_ref[...], axis=1, keepdims=True) / _N


def _run(input, wg, interpret=False):
    tri = (jax.lax.broadcasted_iota(jnp.int32, (_B, _B), 0) <=
           jax.lax.broadcasted_iota(jnp.int32, (_B, _B), 1)).astype(jnp.bfloat16)
    g1, idx, loc, me, laux, ent = pl.pallas_call(
        _router_body,
        grid=(_G,),
        in_specs=[
            pl.BlockSpec((_B, _D), lambda i: (i, 0)),
            pl.BlockSpec((_E, _D), lambda i: (0, 0)),
            pl.BlockSpec((_B, _B), lambda i: (0, 0)),
        ],
        out_specs=[
            pl.BlockSpec((1, 1, _B), lambda i: (i, 0, 0)),
            pl.BlockSpec((1, 1, _B), lambda i: (i, 0, 0)),
            pl.BlockSpec((1, 1, _B), lambda i: (i, 0, 0)),
            pl.BlockSpec((_E, 1), lambda i: (0, 0)),
            pl.BlockSpec((1, 1), lambda i: (0, 0)),
            pl.BlockSpec((1, 1), lambda i: (0, 0)),
        ],
        out_shape=[
            jax.ShapeDtypeStruct((_G, 1, _B), jnp.float32),
            jax.ShapeDtypeStruct((_G, 1, _B), jnp.int32),
            jax.ShapeDtypeStruct((_G, 1, _B), jnp.int32),
            jax.ShapeDtypeStruct((_E, 1), jnp.float32),
            jax.ShapeDtypeStruct((1, 1), jnp.float32),
            jax.ShapeDtypeStruct((1, 1), jnp.float32),
        ],
        scratch_shapes=[
            pltpu.VMEM((_E, 1), jnp.float32),
            pltpu.VMEM((_E, _B), jnp.float32),
            pltpu.VMEM((1, _B), jnp.float32),
        ],
        compiler_params=pltpu.CompilerParams(
            dimension_semantics=("arbitrary",),
        ),
        interpret=interpret,
    )(input, wg, tri)
    return (laux.reshape(()), g1.reshape(_N), idx.reshape(_N),
            loc.reshape(_N), ent.reshape(()))


def kernel(input, wg):
    return _run(input, wg)


# speedup vs baseline: 1.1044x; 1.0617x over previous
import jax
import jax.numpy as jnp
from jax.experimental import pallas as pl
from jax.experimental.pallas import tpu as pltpu

_N = 32768
_D = 768
_E = 64
_B = 1024
_G = _N // _B
_W = 256


def _router_body(x_ref, wg_ref, tri_ref,
                 g1_ref, idx_ref, loc_ref, me_ref, laux_ref, ent_ref,
                 cnt_ref, meacc_ref, entacc_ref):
    i = pl.program_id(0)

    @pl.when(i == 0)
    def _():
        cnt_ref[...] = jnp.zeros_like(cnt_ref)
        meacc_ref[...] = jnp.zeros_like(meacc_ref)
        entacc_ref[...] = jnp.zeros_like(entacc_ref)

    lt = jax.lax.dot_general(wg_ref[...], x_ref[...],
                             (((1,), (1,)), ((), ())),
                             preferred_element_type=jnp.float32)
    m = jnp.max(lt, axis=0, keepdims=True)
    d = lt - m
    e = jnp.exp(d)
    s = jnp.sum(e, axis=0, keepdims=True)
    t1 = jnp.sum(e * d, axis=0, keepdims=True)
    r = 1.0 / s
    entacc_ref[...] += jnp.log(s) - t1 * r
    g1_ref[...] = r.reshape(1, 1, _B)
    meacc_ref[...] += e * r

    row = jax.lax.broadcasted_iota(jnp.int32, (_E, _B), 0)
    idx = jnp.min(jnp.where(lt == m, row, _E), axis=0, keepdims=True)
    idx_ref[...] = idx.reshape(1, 1, _B)
    onehot = (row == idx).astype(jnp.float32)

    ohb = onehot.astype(jnp.bfloat16)
    triw = tri_ref[...]
    base = cnt_ref[...]
    locs = []
    for j in range(_B // _W):
        ohj = jax.lax.slice(ohb, (0, j * _W), (_E, (j + 1) * _W))
        csj = jax.lax.dot_general(ohj, triw, (((1,), (0,)), ((), ())),
                                  preferred_element_type=jnp.float32)
        ohjf = jax.lax.slice(onehot, (0, j * _W), (_E, (j + 1) * _W))
        locs.append(jnp.sum((csj + base) * ohjf, axis=0, keepdims=True) - 1.0)
        base = base + jax.lax.slice(csj, (0, _W - 1), (_E, _W))
    loc = jnp.concatenate(locs, axis=1)
    loc_ref[...] = loc.astype(jnp.int32).reshape(1, 1, _B)
    cnt_ref[...] = base

    @pl.when(i == _G - 1)
    def _():
        me = jnp.sum(meacc_ref[...], axis=1, keepdims=True)
        me_ref[...] = me
        laux_ref[...] = (jnp.sum(me * cnt_ref[...], axis=0, keepdims=True)
                         * (_E / (_N * _N)))
        ent_ref[...] = jnp.sum(entacc_ref[...], axis=1, keepdims=True) / _N


def _run(input, wg, interpret=False):
    tri = (jax.lax.broadcasted_iota(jnp.int32, (_W, _W), 0) <=
           jax.lax.broadcasted_iota(jnp.int32, (_W, _W), 1)).astype(jnp.bfloat16)
    g1, idx, loc, me, laux, ent = pl.pallas_call(
        _router_body,
        grid=(_G,),
        in_specs=[
            pl.BlockSpec((_B, _D), lambda i: (i, 0)),
            pl.BlockSpec((_E, _D), lambda i: (0, 0)),
            pl.BlockSpec((_W, _W), lambda i: (0, 0)),
        ],
        out_specs=[
            pl.BlockSpec((1, 1, _B), lambda i: (i, 0, 0)),
            pl.BlockSpec((1, 1, _B), lambda i: (i, 0, 0)),
            pl.BlockSpec((1, 1, _B), lambda i: (i, 0, 0)),
            pl.BlockSpec((_E, 1), lambda i: (0, 0)),
            pl.BlockSpec((1, 1), lambda i: (0, 0)),
            pl.BlockSpec((1, 1), lambda i: (0, 0)),
        ],
        out_shape=[
            jax.ShapeDtypeStruct((_G, 1, _B), jnp.float32),
            jax.ShapeDtypeStruct((_G, 1, _B), jnp.int32),
            jax.ShapeDtypeStruct((_G, 1, _B), jnp.int32),
            jax.ShapeDtypeStruct((_E, 1), jnp.float32),
            jax.ShapeDtypeStruct((1, 1), jnp.float32),
            jax.ShapeDtypeStruct((1, 1), jnp.float32),
        ],
        scratch_shapes=[
            pltpu.VMEM((_E, 1), jnp.float32),
            pltpu.VMEM((_E, _B), jnp.float32),
            pltpu.VMEM((1, _B), jnp.float32),
        ],
        compiler_params=pltpu.CompilerParams(
            dimension_semantics=("arbitrary",),
        ),
        interpret=interpret,
    )(input, wg, tri)
    return (laux.reshape(()), g1.reshape(_N), idx.reshape(_N),
            loc.reshape(_N), ent.reshape(()))


def kernel(input, wg):
    return _run(input, wg)


# speedup vs baseline: 1.4071x; 1.2741x over previous
import jax
import jax.numpy as jnp
from jax.experimental import pallas as pl
from jax.experimental.pallas import tpu as pltpu

_N = 32768
_D = 768
_E = 64
_B = 2048
_G = _N // _B
_W = 256


def _router_body(x_ref, wg_ref, tri_ref,
                 g1_ref, idx_ref, loc_ref, me_ref, laux_ref, ent_ref,
                 cnt_ref, meacc_ref, entacc_ref):
    i = pl.program_id(0)

    @pl.when(i == 0)
    def _():
        cnt_ref[...] = jnp.zeros_like(cnt_ref)
        meacc_ref[...] = jnp.zeros_like(meacc_ref)
        entacc_ref[...] = jnp.zeros_like(entacc_ref)

    lt = jax.lax.dot_general(wg_ref[...], x_ref[...],
                             (((1,), (1,)), ((), ())),
                             preferred_element_type=jnp.float32)
    m = jnp.max(lt, axis=0, keepdims=True)
    d = lt - m
    e = jnp.exp(d)
    s = jnp.sum(e, axis=0, keepdims=True)
    t1 = jnp.sum(e * d, axis=0, keepdims=True)
    r = 1.0 / s
    entacc_ref[...] += jnp.log(s) - t1 * r
    g1_ref[...] = r.reshape(1, 1, _B)
    meacc_ref[...] += e * r

    row = jax.lax.broadcasted_iota(jnp.int32, (_E, _B), 0)
    idx = jnp.min(jnp.where(lt == m, row, _E), axis=0, keepdims=True)
    idx_ref[...] = idx.reshape(1, 1, _B)
    onehot = (row == idx).astype(jnp.float32)

    ohb = onehot.astype(jnp.bfloat16)
    triw = tri_ref[...]
    base = cnt_ref[...]
    locs = []
    for j in range(_B // _W):
        ohj = jax.lax.slice(ohb, (0, j * _W), (_E, (j + 1) * _W))
        csj = jax.lax.dot_general(ohj, triw, (((1,), (0,)), ((), ())),
                                  preferred_element_type=jnp.float32)
        ohjf = jax.lax.slice(onehot, (0, j * _W), (_E, (j + 1) * _W))
        locs.append(jnp.sum((csj + base) * ohjf, axis=0, keepdims=True) - 1.0)
        base = base + jax.lax.slice(csj, (0, _W - 1), (_E, _W))
    loc = jnp.concatenate(locs, axis=1)
    loc_ref[...] = loc.astype(jnp.int32).reshape(1, 1, _B)
    cnt_ref[...] = base

    @pl.when(i == _G - 1)
    def _():
        me = jnp.sum(meacc_ref[...], axis=1, keepdims=True)
        me_ref[...] = me
        laux_ref[...] = (jnp.sum(me * cnt_ref[...], axis=0, keepdims=True)
                         * (_E / (_N * _N)))
        ent_ref[...] = jnp.sum(entacc_ref[...], axis=1, keepdims=True) / _N


def _run(input, wg, interpret=False):
    tri = (jax.lax.broadcasted_iota(jnp.int32, (_W, _W), 0) <=
           jax.lax.broadcasted_iota(jnp.int32, (_W, _W), 1)).astype(jnp.bfloat16)
    g1, idx, loc, me, laux, ent = pl.pallas_call(
        _router_body,
        grid=(_G,),
        in_specs=[
            pl.BlockSpec((_B, _D), lambda i: (i, 0)),
            pl.BlockSpec((_E, _D), lambda i: (0, 0)),
            pl.BlockSpec((_W, _W), lambda i: (0, 0)),
        ],
        out_specs=[
            pl.BlockSpec((1, 1, _B), lambda i: (i, 0, 0)),
            pl.BlockSpec((1, 1, _B), lambda i: (i, 0, 0)),
            pl.BlockSpec((1, 1, _B), lambda i: (i, 0, 0)),
            pl.BlockSpec((_E, 1), lambda i: (0, 0)),
            pl.BlockSpec((1, 1), lambda i: (0, 0)),
            pl.BlockSpec((1, 1), lambda i: (0, 0)),
        ],
        out_shape=[
            jax.ShapeDtypeStruct((_G, 1, _B), jnp.float32),
            jax.ShapeDtypeStruct((_G, 1, _B), jnp.int32),
            jax.ShapeDtypeStruct((_G, 1, _B), jnp.int32),
            jax.ShapeDtypeStruct((_E, 1), jnp.float32),
            jax.ShapeDtypeStruct((1, 1), jnp.float32),
            jax.ShapeDtypeStruct((1, 1), jnp.float32),
        ],
        scratch_shapes=[
            pltpu.VMEM((_E, 1), jnp.float32),
            pltpu.VMEM((_E, _B), jnp.float32),
            pltpu.VMEM((1, _B), jnp.float32),
        ],
        compiler_params=pltpu.CompilerParams(
            dimension_semantics=("arbitrary",),
        ),
        interpret=interpret,
    )(input, wg, tri)
    return (laux.reshape(()), g1.reshape(_N), idx.reshape(_N),
            loc.reshape(_N), ent.reshape(()))


def kernel(input, wg):
    return _run(input, wg)


# speedup vs baseline: 1.5582x; 1.1074x over previous
import jax
import jax.numpy as jnp
from jax.experimental import pallas as pl
from jax.experimental.pallas import tpu as pltpu

_N = 32768
_D = 768
_E = 64
_B = 4096
_G = _N // _B
_W = 256


def _router_body(x_ref, wg_ref, tri_ref,
                 g1_ref, idx_ref, loc_ref, me_ref, laux_ref, ent_ref,
                 cnt_ref, meacc_ref, entacc_ref):
    i = pl.program_id(0)

    @pl.when(i == 0)
    def _():
        cnt_ref[...] = jnp.zeros_like(cnt_ref)
        meacc_ref[...] = jnp.zeros_like(meacc_ref)
        entacc_ref[...] = jnp.zeros_like(entacc_ref)

    lt = jax.lax.dot_general(wg_ref[...], x_ref[...],
                             (((1,), (1,)), ((), ())),
                             preferred_element_type=jnp.float32)
    m = jnp.max(lt, axis=0, keepdims=True)
    d = lt - m
    e = jnp.exp(d)
    s = jnp.sum(e, axis=0, keepdims=True)
    t1 = jnp.sum(e * d, axis=0, keepdims=True)
    r = 1.0 / s
    entacc_ref[...] += jnp.log(s) - t1 * r
    g1_ref[...] = r.reshape(1, 1, _B)
    meacc_ref[...] += e * r

    row = jax.lax.broadcasted_iota(jnp.int32, (_E, _B), 0)
    idx = jnp.min(jnp.where(lt == m, row, _E), axis=0, keepdims=True)
    idx_ref[...] = idx.reshape(1, 1, _B)
    onehot = (row == idx).astype(jnp.float32)

    ohb = onehot.astype(jnp.bfloat16)
    triw = tri_ref[...]
    base = cnt_ref[...]
    locs = []
    for j in range(_B // _W):
        ohj = jax.lax.slice(ohb, (0, j * _W), (_E, (j + 1) * _W))
        csj = jax.lax.dot_general(ohj, triw, (((1,), (0,)), ((), ())),
                                  preferred_element_type=jnp.float32)
        ohjf = jax.lax.slice(onehot, (0, j * _W), (_E, (j + 1) * _W))
        locs.append(jnp.sum((csj + base) * ohjf, axis=0, keepdims=True) - 1.0)
        base = base + jax.lax.slice(csj, (0, _W - 1), (_E, _W))
    loc = jnp.concatenate(locs, axis=1)
    loc_ref[...] = loc.astype(jnp.int32).reshape(1, 1, _B)
    cnt_ref[...] = base

    @pl.when(i == _G - 1)
    def _():
        me = jnp.sum(meacc_ref[...], axis=1, keepdims=True)
        me_ref[...] = me
        laux_ref[...] = (jnp.sum(me * cnt_ref[...], axis=0, keepdims=True)
                         * (_E / (_N * _N)))
        ent_ref[...] = jnp.sum(entacc_ref[...], axis=1, keepdims=True) / _N


def _run(input, wg, interpret=False):
    tri = (jax.lax.broadcasted_iota(jnp.int32, (_W, _W), 0) <=
           jax.lax.broadcasted_iota(jnp.int32, (_W, _W), 1)).astype(jnp.bfloat16)
    g1, idx, loc, me, laux, ent = pl.pallas_call(
        _router_body,
        grid=(_G,),
        in_specs=[
            pl.BlockSpec((_B, _D), lambda i: (i, 0)),
            pl.BlockSpec((_E, _D), lambda i: (0, 0)),
            pl.BlockSpec((_W, _W), lambda i: (0, 0)),
        ],
        out_specs=[
            pl.BlockSpec((1, 1, _B), lambda i: (i, 0, 0)),
            pl.BlockSpec((1, 1, _B), lambda i: (i, 0, 0)),
            pl.BlockSpec((1, 1, _B), lambda i: (i, 0, 0)),
            pl.BlockSpec((_E, 1), lambda i: (0, 0)),
            pl.BlockSpec((1, 1), lambda i: (0, 0)),
            pl.BlockSpec((1, 1), lambda i: (0, 0)),
        ],
        out_shape=[
            jax.ShapeDtypeStruct((_G, 1, _B), jnp.float32),
            jax.ShapeDtypeStruct((_G, 1, _B), jnp.int32),
            jax.ShapeDtypeStruct((_G, 1, _B), jnp.int32),
            jax.ShapeDtypeStruct((_E, 1), jnp.float32),
            jax.ShapeDtypeStruct((1, 1), jnp.float32),
            jax.ShapeDtypeStruct((1, 1), jnp.float32),
        ],
        scratch_shapes=[
            pltpu.VMEM((_E, 1), jnp.float32),
            pltpu.VMEM((_E, _B), jnp.float32),
            pltpu.VMEM((1, _B), jnp.float32),
        ],
        compiler_params=pltpu.CompilerParams(
            dimension_semantics=("arbitrary",),
        ),
        interpret=interpret,
    )(input, wg, tri)
    return (laux.reshape(()), g1.reshape(_N), idx.reshape(_N),
            loc.reshape(_N), ent.reshape(()))


def kernel(input, wg):
    return _run(input, wg)
